# Initial kernel scaffold; baseline (speedup 1.0000x reference)
import functools

import jax
import jax.numpy as jnp
from jax import lax
from jax.experimental import pallas as pl
from jax.experimental.pallas import tpu as pltpu
from jax.experimental.pallas import tpu_sc as plsc

N_NODES = 10000
N_EDGES = 320000
D = 128
HD = 64
NC = 2
NS = 16
EPT = N_EDGES // NS
CH = 80
NCHUNK = EPT // CH
RPT = N_NODES // NS
ZR = 125
HR = N_NODES // 16


def _leaky(x):
    return jnp.where(x >= 0, x, 0.2 * x)


def _pool_body(v_ref, aw_ref, ab_ref, bw_ref, bb_ref, out_ref):
    x = _leaky(v_ref[...])
    h = _leaky(jnp.dot(x, aw_ref[...], preferred_element_type=jnp.float32)
               + ab_ref[...])
    y = jnp.dot(h, bw_ref[...], preferred_element_type=jnp.float32) + bb_ref[...]
    out_ref[0, :, :] = y[:, :HD]
    out_ref[1, :, :] = y[:, HD:]


def _pool(V, aw, ab, bw, bb):
    blk = 1000
    return pl.pallas_call(
        _pool_body,
        grid=(N_NODES // blk,),
        in_specs=[
            pl.BlockSpec((blk, D), lambda i: (i, 0)),
            pl.BlockSpec((D, D), lambda i: (0, 0)),
            pl.BlockSpec((1, D), lambda i: (0, 0)),
            pl.BlockSpec((D, D), lambda i: (0, 0)),
            pl.BlockSpec((1, D), lambda i: (0, 0)),
        ],
        out_specs=pl.BlockSpec((NC, blk, HD), lambda i: (0, i, 0)),
        out_shape=jax.ShapeDtypeStruct((NC, N_NODES, HD), jnp.float32),
    )(V, aw, ab, bw, bb)


def _mlp_body(e_ref, w0_ref, b0_ref, w1_ref, b1_ref, wb_ref, bb_ref,
              wc_ref, bc_ref, out_ref):
    e = e_ref[...]
    h = jnp.maximum(jnp.dot(e, w0_ref[...], preferred_element_type=jnp.float32)
                    + b0_ref[...], 0.0)
    h = jnp.dot(h, w1_ref[...], preferred_element_type=jnp.float32) + b1_ref[...]
    scale = jax.nn.sigmoid(
        jnp.dot(h, wb_ref[...], preferred_element_type=jnp.float32) + bb_ref[...])
    shift = jnp.dot(h, wc_ref[...], preferred_element_type=jnp.float32) + bc_ref[...]
    out_ref[0, :, 0:HD] = scale[:, :HD]
    out_ref[0, :, HD:D] = shift[:, :HD]
    out_ref[1, :, 0:HD] = scale[:, HD:]
    out_ref[1, :, HD:D] = shift[:, HD:]


def _edge_mlp(E, w0, b0, w1, b1, wb, bb, wc, bc):
    blk = 4000
    ed = E.shape[1]
    return pl.pallas_call(
        _mlp_body,
        grid=(N_EDGES // blk,),
        in_specs=[
            pl.BlockSpec((blk, ed), lambda i: (i, 0)),
            pl.BlockSpec((ed, ed), lambda i: (0, 0)),
            pl.BlockSpec((1, ed), lambda i: (0, 0)),
            pl.BlockSpec((ed, ed), lambda i: (0, 0)),
            pl.BlockSpec((1, ed), lambda i: (0, 0)),
            pl.BlockSpec((ed, D), lambda i: (0, 0)),
            pl.BlockSpec((1, D), lambda i: (0, 0)),
            pl.BlockSpec((ed, D), lambda i: (0, 0)),
            pl.BlockSpec((1, D), lambda i: (0, 0)),
        ],
        out_specs=pl.BlockSpec((NC, blk, D), lambda i: (0, i, 0)),
        out_shape=jax.ShapeDtypeStruct((NC, N_EDGES, D), jnp.float32),
    )(E, w0, b0, w1, b1, wb, bb, wc, bc)


def _sc_body(tab, coef, src, dst, v1o, v2o, co,
             s_idx, d_idx, cbuf, gbuf, m1b, m2b, hist, zbuf, acc1, acc2, sem):
    c = lax.axis_index("c")
    s = lax.axis_index("s")

    zeros16 = jnp.zeros((16,), jnp.float32)

    def _zrow(i, carry):
        for j in range(HD // 16):
            zbuf[i, pl.ds(j * 16, 16)] = zeros16
        return carry
    lax.fori_loop(0, ZR, _zrow, 0)

    def _zhist(i, carry):
        hist[i, :] = zeros16
        return carry
    lax.fori_loop(0, HR, _zhist, 0)

    for k in range(RPT // ZR):
        r0 = s * RPT + k * ZR
        pltpu.sync_copy(zbuf, acc1.at[pl.ds(r0, ZR), :])
        pltpu.sync_copy(zbuf, acc2.at[pl.ds(r0, ZR), :])
    plsc.subcore_barrier()

    tab_off = c * N_NODES
    coef_off = c * N_EDGES
    ones16 = jnp.full((16,), 1.0, jnp.float32)

    def _chunk(i, carry):
        b = s * EPT + i * CH
        pltpu.sync_copy(src.at[pl.ds(b, CH)], s_idx)
        pltpu.sync_copy(dst.at[pl.ds(b, CH)], d_idx)
        pltpu.sync_copy(coef.at[pl.ds(coef_off + b, CH), :], cbuf)
        for j in range(CH // 16):
            sl = pl.ds(j * 16, 16)
            s_idx[sl] = s_idx[sl] + tab_off
        pltpu.async_copy(tab.at[s_idx], gbuf, sem).wait()

        def _row(r, cc):
            for j in range(HD // 16):
                sl = pl.ds(j * 16, 16)
                sc16 = cbuf[r, sl]
                sh16 = cbuf[r, pl.ds(HD + j * 16, 16)]
                m1 = sc16 * gbuf[r, sl] + sh16
                m1b[r, sl] = m1
                m2b[r, sl] = m1 * m1
            return cc
        lax.fori_loop(0, CH, _row, 0)

        for j in range(CH // 16):
            dj = d_idx[pl.ds(j * 16, 16)]
            rr = lax.shift_right_logical(dj, 4)
            ll = lax.bitwise_and(dj, 15)
            plsc.addupdate_scatter(hist, [rr, ll], ones16)

        pltpu.sync_copy(m1b, acc1.at[d_idx], add=True)
        pltpu.sync_copy(m2b, acc2.at[d_idx], add=True)
        return carry
    lax.fori_loop(0, NCHUNK, _chunk, 0)

    plsc.subcore_barrier()

    for k in range(RPT // ZR):
        r0 = s * RPT + k * ZR
        pltpu.sync_copy(acc1.at[pl.ds(r0, ZR), :], zbuf)
        pltpu.sync_copy(zbuf, v1o.at[pl.ds(tab_off + r0, ZR), :])
        pltpu.sync_copy(acc2.at[pl.ds(r0, ZR), :], zbuf)
        pltpu.sync_copy(zbuf, v2o.at[pl.ds(tab_off + r0, ZR), :])
    wid = c * NS + s
    pltpu.sync_copy(hist, co.at[pl.ds(wid * HR, HR), :])


def _sc_scatter(tab, coef, src, dst):
    mesh = plsc.VectorSubcoreMesh(core_axis_name="c", subcore_axis_name="s")
    kern = pl.kernel(
        _sc_body,
        mesh=mesh,
        out_type=(
            jax.ShapeDtypeStruct((NC * N_NODES, HD), jnp.float32),
            jax.ShapeDtypeStruct((NC * N_NODES, HD), jnp.float32),
            jax.ShapeDtypeStruct((NC * NS * HR, 16), jnp.float32),
        ),
        scratch_types=[
            pltpu.VMEM((CH,), jnp.int32),
            pltpu.VMEM((CH,), jnp.int32),
            pltpu.VMEM((CH, D), jnp.float32),
            pltpu.VMEM((CH, HD), jnp.float32),
            pltpu.VMEM((CH, HD), jnp.float32),
            pltpu.VMEM((CH, HD), jnp.float32),
            pltpu.VMEM((HR, 16), jnp.float32),
            pltpu.VMEM((ZR, HD), jnp.float32),
            pltpu.VMEM_SHARED((N_NODES, HD), jnp.float32),
            pltpu.VMEM_SHARED((N_NODES, HD), jnp.float32),
            pltpu.SemaphoreType.DMA,
        ],
    )
    return kern(tab, coef, src, dst)


def _epi_body(v1_ref, v2_ref, cnt_ref, out_ref):
    cnt = jnp.sum(cnt_ref[...], axis=0)
    denom = jnp.maximum(cnt, 1.0)[:, None]
    s1 = v1_ref[...]
    s2 = v2_ref[...]
    lo = jnp.sqrt(jnp.maximum(s2[0] - s1[0], 0.0) / denom + 1e-5)
    hi = jnp.sqrt(jnp.maximum(s2[1] - s1[1], 0.0) / denom + 1e-5)
    out_ref[:, :HD] = lo
    out_ref[:, HD:] = hi


def _epilogue(v1, v2, cntf):
    blk = 1250
    return pl.pallas_call(
        _epi_body,
        grid=(N_NODES // blk,),
        in_specs=[
            pl.BlockSpec((NC, blk, HD), lambda i: (0, i, 0)),
            pl.BlockSpec((NC, blk, HD), lambda i: (0, i, 0)),
            pl.BlockSpec((NS, blk), lambda i: (0, i)),
        ],
        out_specs=pl.BlockSpec((blk, D), lambda i: (i, 0)),
        out_shape=jax.ShapeDtypeStruct((N_NODES, D), jnp.float32),
    )(v1, v2, cntf)


def kernel(V, E, edge_index, pool_A_w, pool_A_b, pool_B_w, pool_B_b,
           mlp0_w, mlp0_b, mlp1_w, mlp1_b, lin_B_w, lin_B_b, lin_C_w, lin_C_b):
    src = edge_index[0]
    dst = edge_index[1]
    tab = _pool(V, pool_A_w, pool_A_b.reshape(1, D),
                pool_B_w, pool_B_b.reshape(1, D)).reshape(NC * N_NODES, HD)
    coef = _edge_mlp(E, mlp0_w, mlp0_b.reshape(1, -1), mlp1_w,
                     mlp1_b.reshape(1, -1), lin_B_w, lin_B_b.reshape(1, D),
                     lin_C_w, lin_C_b.reshape(1, D)).reshape(NC * N_EDGES, D)
    v1f, v2f, cnt = _sc_scatter(tab, coef, src, dst)
    v1 = v1f.reshape(NC, N_NODES, HD)
    v2 = v2f.reshape(NC, N_NODES, HD)
    cntf = cnt.reshape(NC, NS, N_NODES)[0]
    return _epilogue(v1, v2, cntf)


# speedup vs baseline: 1.8462x; 1.8462x over previous
import functools

import jax
import jax.numpy as jnp
from jax import lax
from jax.experimental import pallas as pl
from jax.experimental.pallas import tpu as pltpu
from jax.experimental.pallas import tpu_sc as plsc

N_NODES = 10000
N_EDGES = 320000
D = 128
HD = 64
NC = 2
NS = 16
EPT = N_EDGES // NS
CH = 80
NCHUNK = EPT // CH
OWN = 624
ZR = 208
AW = 80


def _leaky(x):
    return jnp.where(x >= 0, x, 0.2 * x)


def _pool_body(v_ref, aw_ref, ab_ref, bw_ref, bb_ref, out_ref):
    x = _leaky(v_ref[...])
    h = _leaky(jnp.dot(x, aw_ref[...], preferred_element_type=jnp.float32)
               + ab_ref[...])
    y = jnp.dot(h, bw_ref[...], preferred_element_type=jnp.float32) + bb_ref[...]
    out_ref[0, :, :] = y[:, :HD]
    out_ref[1, :, :] = y[:, HD:]


def _pool(V, aw, ab, bw, bb):
    blk = 1000
    return pl.pallas_call(
        _pool_body,
        grid=(N_NODES // blk,),
        in_specs=[
            pl.BlockSpec((blk, D), lambda i: (i, 0)),
            pl.BlockSpec((D, D), lambda i: (0, 0)),
            pl.BlockSpec((1, D), lambda i: (0, 0)),
            pl.BlockSpec((D, D), lambda i: (0, 0)),
            pl.BlockSpec((1, D), lambda i: (0, 0)),
        ],
        out_specs=pl.BlockSpec((NC, blk, HD), lambda i: (0, i, 0)),
        out_shape=jax.ShapeDtypeStruct((NC, N_NODES, HD), jnp.float32),
    )(V, aw, ab, bw, bb)


def _mlp_body(e_ref, w0_ref, b0_ref, w1_ref, b1_ref, wb_ref, bb_ref,
              wc_ref, bc_ref, out_ref):
    e = e_ref[...]
    h = jnp.maximum(jnp.dot(e, w0_ref[...], preferred_element_type=jnp.float32)
                    + b0_ref[...], 0.0)
    h = jnp.dot(h, w1_ref[...], preferred_element_type=jnp.float32) + b1_ref[...]
    scale = jax.nn.sigmoid(
        jnp.dot(h, wb_ref[...], preferred_element_type=jnp.float32) + bb_ref[...])
    shift = jnp.dot(h, wc_ref[...], preferred_element_type=jnp.float32) + bc_ref[...]
    out_ref[0, :, 0:HD] = scale[:, :HD]
    out_ref[0, :, HD:D] = shift[:, :HD]
    out_ref[1, :, 0:HD] = scale[:, HD:]
    out_ref[1, :, HD:D] = shift[:, HD:]


def _edge_mlp(E, w0, b0, w1, b1, wb, bb, wc, bc):
    blk = 4000
    ed = E.shape[1]
    return pl.pallas_call(
        _mlp_body,
        grid=(N_EDGES // blk,),
        in_specs=[
            pl.BlockSpec((blk, ed), lambda i: (i, 0)),
            pl.BlockSpec((ed, ed), lambda i: (0, 0)),
            pl.BlockSpec((1, ed), lambda i: (0, 0)),
            pl.BlockSpec((ed, ed), lambda i: (0, 0)),
            pl.BlockSpec((1, ed), lambda i: (0, 0)),
            pl.BlockSpec((ed, D), lambda i: (0, 0)),
            pl.BlockSpec((1, D), lambda i: (0, 0)),
            pl.BlockSpec((ed, D), lambda i: (0, 0)),
            pl.BlockSpec((1, D), lambda i: (0, 0)),
        ],
        out_specs=pl.BlockSpec((NC, blk, D), lambda i: (0, i, 0)),
        out_shape=jax.ShapeDtypeStruct((NC, N_EDGES, D), jnp.float32),
    )(E, w0, b0, w1, b1, wb, bb, wc, bc)


def _sc_body(tab, coef, src, dst, v1o, v2o,
             s_idx, d_idx, cbuf, gbuf, m1b, m2b, acc1, acc2, sem):
    c = lax.axis_index("c")
    s = lax.axis_index("s")

    zeros16 = jnp.zeros((16,), jnp.float32)
    cnt16 = jnp.where(lax.iota(jnp.int32, 16) == 0, 1.0, 0.0).astype(jnp.float32)

    def _zrow(i, carry):
        for j in range(AW // 16):
            m1b[i, pl.ds(j * 16, 16)] = zeros16
        for j in range(HD // 16):
            m2b[i, pl.ds(j * 16, 16)] = zeros16
        return carry
    lax.fori_loop(0, CH, _zrow, 0)

    for k in range(OWN // CH):
        r0 = pl.multiple_of(s * OWN + k * CH, 8)
        pltpu.sync_copy(m1b, acc1.at[pl.ds(r0, CH), :])
        pltpu.sync_copy(m2b, acc2.at[pl.ds(r0, CH), :])
    rz = pl.multiple_of(s * OWN + (OWN // CH) * CH, 8)
    pltpu.sync_copy(m1b.at[pl.ds(0, OWN % CH), :], acc1.at[pl.ds(rz, OWN % CH), :])
    pltpu.sync_copy(m2b.at[pl.ds(0, OWN % CH), :], acc2.at[pl.ds(rz, OWN % CH), :])
    @pl.when(s == NS - 1)
    def _ztail():
        pltpu.sync_copy(m1b.at[pl.ds(0, 16), :], acc1.at[pl.ds(NS * OWN, 16), :])
        pltpu.sync_copy(m2b.at[pl.ds(0, 16), :], acc2.at[pl.ds(NS * OWN, 16), :])

    def _ztail_m1(i, carry):
        m1b[i, pl.ds(HD, 16)] = cnt16
        return carry
    lax.fori_loop(0, CH, _ztail_m1, 0)
    plsc.subcore_barrier()

    tab_off = c * N_NODES
    coef_off = c * N_EDGES

    def _chunk(i, carry):
        b = pl.multiple_of(s * EPT + i * CH, 8)
        pltpu.sync_copy(src.at[pl.ds(b, CH)], s_idx)
        pltpu.sync_copy(dst.at[pl.ds(b, CH)], d_idx)
        pltpu.sync_copy(coef.at[pl.ds(pl.multiple_of(coef_off + b, 8), CH), :],
                        cbuf)
        for j in range(CH // 16):
            sl = pl.ds(j * 16, 16)
            s_idx[sl] = s_idx[sl] + tab_off
        pltpu.async_copy(tab.at[s_idx], gbuf, sem).wait()

        def _row(r, cc):
            for j in range(HD // 16):
                sl = pl.ds(j * 16, 16)
                sc16 = cbuf[r, sl]
                sh16 = cbuf[r, pl.ds(HD + j * 16, 16)]
                m1 = sc16 * gbuf[r, sl] + sh16
                m1b[r, sl] = m1
                m2b[r, sl] = m1 * m1
            return cc
        lax.fori_loop(0, CH, _row, 0)

        pltpu.sync_copy(m1b, acc1.at[d_idx], add=True)
        pltpu.sync_copy(m2b, acc2.at[d_idx], add=True)
        return carry
    lax.fori_loop(0, NCHUNK, _chunk, 0)

    plsc.subcore_barrier()

    for k in range(OWN // CH):
        r0 = pl.multiple_of(s * OWN + k * CH, 8)
        ro = pl.multiple_of(tab_off + r0, 8)
        pltpu.sync_copy(acc1.at[pl.ds(r0, CH), :], m1b)
        pltpu.sync_copy(m1b, v1o.at[pl.ds(ro, CH), :])
        pltpu.sync_copy(acc2.at[pl.ds(r0, CH), :], m2b)
        pltpu.sync_copy(m2b, v2o.at[pl.ds(ro, CH), :])
    TL = OWN % CH
    rz2 = pl.multiple_of(s * OWN + (OWN // CH) * CH, 8)
    ro2 = pl.multiple_of(tab_off + rz2, 8)
    pltpu.sync_copy(acc1.at[pl.ds(rz2, TL), :], m1b.at[pl.ds(0, TL), :])
    pltpu.sync_copy(m1b.at[pl.ds(0, TL), :], v1o.at[pl.ds(ro2, TL), :])
    pltpu.sync_copy(acc2.at[pl.ds(rz2, TL), :], m2b.at[pl.ds(0, TL), :])
    pltpu.sync_copy(m2b.at[pl.ds(0, TL), :], v2o.at[pl.ds(ro2, TL), :])
    @pl.when(s == NS - 1)
    def _tail():
        rt = pl.multiple_of(tab_off + NS * OWN, 8)
        pltpu.sync_copy(acc1.at[pl.ds(NS * OWN, 16), :], m1b.at[pl.ds(0, 16), :])
        pltpu.sync_copy(m1b.at[pl.ds(0, 16), :], v1o.at[pl.ds(rt, 16), :])
        pltpu.sync_copy(acc2.at[pl.ds(NS * OWN, 16), :], m2b.at[pl.ds(0, 16), :])
        pltpu.sync_copy(m2b.at[pl.ds(0, 16), :], v2o.at[pl.ds(rt, 16), :])


def _sc_scatter(tab, coef, src, dst):
    mesh = plsc.VectorSubcoreMesh(core_axis_name="c", subcore_axis_name="s")
    kern = pl.kernel(
        _sc_body,
        mesh=mesh,
        compiler_params=pltpu.CompilerParams(use_tc_tiling_on_sc=False),
        out_type=(
            jax.ShapeDtypeStruct((NC * N_NODES, AW), jnp.float32),
            jax.ShapeDtypeStruct((NC * N_NODES, HD), jnp.float32),
        ),
        scratch_types=[
            pltpu.VMEM((CH,), jnp.int32),
            pltpu.VMEM((CH,), jnp.int32),
            pltpu.VMEM((CH, D), jnp.float32),
            pltpu.VMEM((CH, HD), jnp.float32),
            pltpu.VMEM((CH, AW), jnp.float32),
            pltpu.VMEM((CH, HD), jnp.float32),
            pltpu.VMEM_SHARED((N_NODES, AW), jnp.float32),
            pltpu.VMEM_SHARED((N_NODES, HD), jnp.float32),
            pltpu.SemaphoreType.DMA,
        ],
    )
    return kern(tab, coef, src, dst)


def _epi_body(v1_ref, v2_ref, out_ref):
    s1 = v1_ref[...]
    s2 = v2_ref[...]
    denom = jnp.maximum(s1[0, :, HD:HD + 1], 1.0)
    lo = jnp.sqrt(jnp.maximum(s2[0] - s1[0, :, :HD], 0.0) / denom + 1e-5)
    hi = jnp.sqrt(jnp.maximum(s2[1] - s1[1, :, :HD], 0.0) / denom + 1e-5)
    out_ref[:, :HD] = lo
    out_ref[:, HD:] = hi


def _epilogue(v1, v2):
    return pl.pallas_call(
        _epi_body,
        out_shape=jax.ShapeDtypeStruct((N_NODES, D), jnp.float32),
    )(v1, v2)


def kernel(V, E, edge_index, pool_A_w, pool_A_b, pool_B_w, pool_B_b,
           mlp0_w, mlp0_b, mlp1_w, mlp1_b, lin_B_w, lin_B_b, lin_C_w, lin_C_b):
    src = edge_index[0]
    dst = edge_index[1]
    tab = _pool(V, pool_A_w, pool_A_b.reshape(1, D),
                pool_B_w, pool_B_b.reshape(1, D)).reshape(NC * N_NODES, HD)
    coef = _edge_mlp(E, mlp0_w, mlp0_b.reshape(1, -1), mlp1_w,
                     mlp1_b.reshape(1, -1), lin_B_w, lin_B_b.reshape(1, D),
                     lin_C_w, lin_C_b.reshape(1, D)).reshape(NC * N_EDGES, D)
    v1f, v2f = _sc_scatter(tab, coef, src, dst)
    return _epilogue(v1f.reshape(NC, N_NODES, AW), v2f.reshape(NC, N_NODES, HD))


# speedup vs baseline: 3.1969x; 1.7316x over previous
import functools

import jax
import jax.numpy as jnp
from jax import lax
from jax.experimental import pallas as pl
from jax.experimental.pallas import tpu as pltpu
from jax.experimental.pallas import tpu_sc as plsc

N_NODES = 10000
N_EDGES = 320000
D = 128
HD = 64
NC = 2
NS = 16
EPT = N_EDGES // NS
CH = 40
NCHUNK = EPT // CH
OWN = 624
MW = 144


def _leaky(x):
    return jnp.where(x >= 0, x, 0.2 * x)


def _pool_body(v_ref, aw_ref, ab_ref, bw_ref, bb_ref, out_ref):
    x = _leaky(v_ref[...])
    h = _leaky(jnp.dot(x, aw_ref[...], preferred_element_type=jnp.float32)
               + ab_ref[...])
    y = jnp.dot(h, bw_ref[...], preferred_element_type=jnp.float32) + bb_ref[...]
    out_ref[0, :, :] = y[:, :HD]
    out_ref[1, :, :] = y[:, HD:]


def _pool(V, aw, ab, bw, bb):
    blk = 1000
    return pl.pallas_call(
        _pool_body,
        grid=(N_NODES // blk,),
        in_specs=[
            pl.BlockSpec((blk, D), lambda i: (i, 0)),
            pl.BlockSpec((D, D), lambda i: (0, 0)),
            pl.BlockSpec((1, D), lambda i: (0, 0)),
            pl.BlockSpec((D, D), lambda i: (0, 0)),
            pl.BlockSpec((1, D), lambda i: (0, 0)),
        ],
        out_specs=pl.BlockSpec((NC, blk, HD), lambda i: (0, i, 0)),
        out_shape=jax.ShapeDtypeStruct((NC, N_NODES, HD), jnp.float32),
    )(V, aw, ab, bw, bb)


def _mlp_body(e_ref, w0_ref, b0_ref, w1_ref, b1_ref, wb_ref, bb_ref,
              wc_ref, bc_ref, out_ref):
    e = e_ref[...]
    h = jnp.maximum(jnp.dot(e, w0_ref[...], preferred_element_type=jnp.float32)
                    + b0_ref[...], 0.0)
    h = jnp.dot(h, w1_ref[...], preferred_element_type=jnp.float32) + b1_ref[...]
    scale = jax.nn.sigmoid(
        jnp.dot(h, wb_ref[...], preferred_element_type=jnp.float32) + bb_ref[...])
    shift = jnp.dot(h, wc_ref[...], preferred_element_type=jnp.float32) + bc_ref[...]
    out_ref[0, :, 0:HD] = scale[:, :HD]
    out_ref[0, :, HD:D] = shift[:, :HD]
    out_ref[1, :, 0:HD] = scale[:, HD:]
    out_ref[1, :, HD:D] = shift[:, HD:]


def _edge_mlp(E, w0, b0, w1, b1, wb, bb, wc, bc):
    blk = 4000
    ed = E.shape[1]
    return pl.pallas_call(
        _mlp_body,
        grid=(N_EDGES // blk,),
        in_specs=[
            pl.BlockSpec((blk, ed), lambda i: (i, 0)),
            pl.BlockSpec((ed, ed), lambda i: (0, 0)),
            pl.BlockSpec((1, ed), lambda i: (0, 0)),
            pl.BlockSpec((ed, ed), lambda i: (0, 0)),
            pl.BlockSpec((1, ed), lambda i: (0, 0)),
            pl.BlockSpec((ed, D), lambda i: (0, 0)),
            pl.BlockSpec((1, D), lambda i: (0, 0)),
            pl.BlockSpec((ed, D), lambda i: (0, 0)),
            pl.BlockSpec((1, D), lambda i: (0, 0)),
        ],
        out_specs=pl.BlockSpec((NC, blk, D), lambda i: (0, i, 0)),
        out_shape=jax.ShapeDtypeStruct((NC, N_EDGES, D), jnp.float32),
    )(E, w0, b0, w1, b1, wb, bb, wc, bc)


def _sc_body(tab, coef, srcA, dst, vo,
             si0, si1, si2, si3, di0, di1, di2, di3,
             cb0, cb1, gb0, gb1, mb0, mb1, acc,
             sg0, sg1, sc0, sc1, ss0, ss1, sif0, sif1, sif2, sif3):
    c = lax.axis_index("c")
    s = lax.axis_index("s")
    sis = [si0, si1, si2, si3]
    dis = [di0, di1, di2, di3]
    cbs = [cb0, cb1]
    gbs = [gb0, gb1]
    mbs = [mb0, mb1]
    sgs = [sg0, sg1]
    scs = [sc0, sc1]
    sss = [ss0, ss1]
    sifs = [sif0, sif1, sif2, sif3]

    zeros16 = jnp.zeros((16,), jnp.float32)
    cnt16 = jnp.where(lax.iota(jnp.int32, 16) == 0, 1.0, 0.0).astype(jnp.float32)

    def _zrow(i, carry):
        for j in range(MW // 16):
            mb0[i, pl.ds(j * 16, 16)] = zeros16
            mb1[i, pl.ds(j * 16, 16)] = zeros16
        mb0[i, pl.ds(2 * HD, 16)] = cnt16
        mb1[i, pl.ds(2 * HD, 16)] = cnt16
        return carry
    lax.fori_loop(0, CH, _zrow, 0)

    def _zc(i, carry):
        mb0[i, pl.ds(2 * HD, 16)] = zeros16
        return carry
    lax.fori_loop(0, CH, _zc, 0)
    for k in range(OWN // CH):
        r0 = pl.multiple_of(s * OWN + k * CH, 8)
        pltpu.sync_copy(mb0, acc.at[pl.ds(r0, CH), :])
    TLZ = OWN % CH
    rz = pl.multiple_of(s * OWN + (OWN // CH) * CH, 8)
    pltpu.sync_copy(mb0.at[pl.ds(0, TLZ), :], acc.at[pl.ds(rz, TLZ), :])
    @pl.when(s == NS - 1)
    def _ztail():
        pltpu.sync_copy(mb0.at[pl.ds(0, 16), :], acc.at[pl.ds(NS * OWN, 16), :])
    def _rc(i, carry):
        mb0[i, pl.ds(2 * HD, 16)] = cnt16
        return carry
    lax.fori_loop(0, CH, _rc, 0)
    plsc.subcore_barrier()

    src_off = c * N_EDGES
    base0 = s * EPT

    def _issue_idx(i, sl):
        b = pl.multiple_of(base0 + i * CH, 8)
        pltpu.async_copy(srcA.at[pl.ds(pl.multiple_of(src_off + b, 8), CH)],
                         sis[sl], sifs[sl])
        pltpu.async_copy(dst.at[pl.ds(b, CH)], dis[sl], sifs[sl])

    def _wait_idx(sl):
        pltpu.make_async_copy(srcA.at[pl.ds(0, CH)], sis[sl], sifs[sl]).wait()
        pltpu.make_async_copy(dst.at[pl.ds(0, CH)], dis[sl], sifs[sl]).wait()

    def _issue_fetch(i, sl4, sl2):
        b = pl.multiple_of(base0 + i * CH, 8)
        pltpu.async_copy(coef.at[pl.ds(pl.multiple_of(src_off + b, 8), CH), :],
                         cbs[sl2], scs[sl2])
        pltpu.async_copy(tab.at[sis[sl4]], gbs[sl2], sgs[sl2])

    def _wait_fetch(sl2):
        pltpu.make_async_copy(coef.at[pl.ds(0, CH), :], cbs[sl2], scs[sl2]).wait()
        pltpu.make_async_copy(tab.at[pl.ds(0, CH), :], gbs[sl2], sgs[sl2]).wait()

    def _compute(sl2):
        cb, gb, mb = cbs[sl2], gbs[sl2], mbs[sl2]

        def _row(r, carry):
            for j in range(HD // 16):
                sl = pl.ds(j * 16, 16)
                m1 = cb[r, sl] * gb[r, sl] + cb[r, pl.ds(HD + j * 16, 16)]
                mb[r, sl] = m1
                mb[r, pl.ds(HD + j * 16, 16)] = m1 * m1
            return carry
        lax.fori_loop(0, CH, _row, 0)

    def _issue_scatter(sl4, sl2):
        pltpu.async_copy(mbs[sl2], acc.at[dis[sl4]], sss[sl2], add=True)

    def _wait_scatter(sl2):
        pltpu.make_async_copy(mbs[sl2], acc.at[pl.ds(0, CH), :], sss[sl2]).wait()

    for i in (0, 1):
        _issue_idx(i, i)
    for i in (0, 1):
        _wait_idx(i)
        _issue_fetch(i, i, i % 2)
    for i in (0, 1, 2, 3):
        if i >= 2:
            _wait_scatter(i % 2)
        _issue_idx(i + 2, (i + 2) % 4)
        _wait_fetch(i % 2)
        _compute(i % 2)
        _issue_scatter(i % 4, i % 2)
        _wait_idx((i + 2) % 4)
        _issue_fetch(i + 2, (i + 2) % 4, i % 2)

    def _quad(k, carry):
        i0 = 4 * k
        for t in range(4):
            i = i0 + t
            _wait_scatter(t % 2)
            _issue_idx(i + 2, (t + 2) % 4)
            _wait_fetch(t % 2)
            _compute(t % 2)
            _issue_scatter(t % 4, t % 2)
            _wait_idx((t + 2) % 4)
            _issue_fetch(i + 2, (t + 2) % 4, t % 2)
        return carry
    lax.fori_loop(1, NCHUNK // 4 - 1, _quad, 0)

    for t in range(4):
        i = NCHUNK - 4 + t
        _wait_scatter(t % 2)
        if i + 2 < NCHUNK:
            _issue_idx(i + 2, (t + 2) % 4)
        _wait_fetch(t % 2)
        _compute(t % 2)
        _issue_scatter(t % 4, t % 2)
        if i + 2 < NCHUNK:
            _wait_idx((t + 2) % 4)
            _issue_fetch(i + 2, (t + 2) % 4, t % 2)
    _wait_scatter(0)
    _wait_scatter(1)

    plsc.subcore_barrier()

    tab_off = c * N_NODES
    for k in range(OWN // CH):
        r0 = pl.multiple_of(s * OWN + k * CH, 8)
        ro = pl.multiple_of(tab_off + r0, 8)
        pltpu.sync_copy(acc.at[pl.ds(r0, CH), :], mb0)
        pltpu.sync_copy(mb0, vo.at[pl.ds(ro, CH), :])
    TL = OWN % CH
    rz2 = pl.multiple_of(s * OWN + (OWN // CH) * CH, 8)
    ro2 = pl.multiple_of(tab_off + rz2, 8)
    pltpu.sync_copy(acc.at[pl.ds(rz2, TL), :], mb0.at[pl.ds(0, TL), :])
    pltpu.sync_copy(mb0.at[pl.ds(0, TL), :], vo.at[pl.ds(ro2, TL), :])
    @pl.when(s == NS - 1)
    def _tail():
        rt = pl.multiple_of(tab_off + NS * OWN, 8)
        pltpu.sync_copy(acc.at[pl.ds(NS * OWN, 16), :], mb0.at[pl.ds(0, 16), :])
        pltpu.sync_copy(mb0.at[pl.ds(0, 16), :], vo.at[pl.ds(rt, 16), :])


def _sc_scatter(tab, coef, srcA, dst):
    mesh = plsc.VectorSubcoreMesh(core_axis_name="c", subcore_axis_name="s")
    kern = pl.kernel(
        _sc_body,
        mesh=mesh,
        compiler_params=pltpu.CompilerParams(use_tc_tiling_on_sc=False),
        out_type=jax.ShapeDtypeStruct((NC * N_NODES, MW), jnp.float32),
        scratch_types=(
            [pltpu.VMEM((CH,), jnp.int32) for _ in range(8)]
            + [pltpu.VMEM((CH, D), jnp.float32) for _ in range(2)]
            + [pltpu.VMEM((CH, HD), jnp.float32) for _ in range(2)]
            + [pltpu.VMEM((CH, MW), jnp.float32) for _ in range(2)]
            + [pltpu.VMEM_SHARED((N_NODES, MW), jnp.float32)]
            + [pltpu.SemaphoreType.DMA for _ in range(10)]
        ),
    )
    return kern(tab, coef, srcA, dst)


def _epi_body(v_ref, out_ref):
    v = v_ref[...]
    denom = jnp.maximum(v[0, :, 2 * HD:2 * HD + 1], 1.0)
    lo = jnp.sqrt(
        jnp.maximum(v[0, :, HD:2 * HD] - v[0, :, :HD], 0.0) / denom + 1e-5)
    hi = jnp.sqrt(
        jnp.maximum(v[1, :, HD:2 * HD] - v[1, :, :HD], 0.0) / denom + 1e-5)
    out_ref[:, :HD] = lo
    out_ref[:, HD:] = hi


def _epilogue(v):
    return pl.pallas_call(
        _epi_body,
        out_shape=jax.ShapeDtypeStruct((N_NODES, D), jnp.float32),
    )(v)


def kernel(V, E, edge_index, pool_A_w, pool_A_b, pool_B_w, pool_B_b,
           mlp0_w, mlp0_b, mlp1_w, mlp1_b, lin_B_w, lin_B_b, lin_C_w, lin_C_b):
    src = edge_index[0]
    dst = edge_index[1]
    srcA = jnp.concatenate([src, src + N_NODES])
    tab = _pool(V, pool_A_w, pool_A_b.reshape(1, D),
                pool_B_w, pool_B_b.reshape(1, D)).reshape(NC * N_NODES, HD)
    coef = _edge_mlp(E, mlp0_w, mlp0_b.reshape(1, -1), mlp1_w,
                     mlp1_b.reshape(1, -1), lin_B_w, lin_B_b.reshape(1, D),
                     lin_C_w, lin_C_b.reshape(1, D)).reshape(NC * N_EDGES, D)
    vo = _sc_scatter(tab, coef, srcA, dst)
    return _epilogue(vo.reshape(NC, N_NODES, MW))
